# baseline (speedup 1.0000x reference)
import dataclasses
import functools

import jax
import jax.numpy as jnp
from jax import lax
from jax.experimental import pallas as pl
from jax.experimental.pallas import tpu as pltpu
from jax.experimental.pallas import tpu_sc as plsc

N_NODES = 10000
N_EDGES = 320000
D = 128

NC = 2
NS = 16
NW = NC * NS

C = 80
EPW = N_EDGES // NW
NCHUNK = EPW // C
NI = 8
NR = 4
RPS = 632


def _sc_aggregate(row, col, val, X):
    mesh = plsc.VectorSubcoreMesh(core_axis_name="c", subcore_axis_name="s")

    cp = pltpu.CompilerParams()
    if "needs_layout_passes" in pltpu.CompilerParams.__dataclass_fields__:
        cp = dataclasses.replace(cp, needs_layout_passes=False)

    @functools.partial(
        pl.kernel,
        out_type=jax.ShapeDtypeStruct((NC, N_NODES, D), jnp.float32),
        mesh=mesh,
        compiler_params=cp,
        scratch_types=(
            [pltpu.VMEM((3 * C,), jnp.int32)] * NI
            + [pltpu.VMEM((C,), jnp.int32)] * NI
            + [pltpu.VMEM((C, D), jnp.float32)] * NR
            + [pltpu.VMEM_SHARED((N_NODES, D), jnp.float32)]
            + [pltpu.SemaphoreType.DMA] * (NI + 2 * NR)
        ),
    )
    def agg(pk_hbm, x_hbm, out_hbm, *refs):
        pk_b = refs[0:NI]
        row_b = refs[NI:2 * NI]
        rows_b = refs[2 * NI:2 * NI + NR]
        acc = refs[2 * NI + NR]
        si = refs[2 * NI + NR + 1:2 * NI + NR + 1 + NI]
        sg = refs[2 * NI + NR + 1 + NI:2 * NI + NR + 1 + NI + NR]
        ss = refs[2 * NI + NR + 1 + NI + NR:]

        cc = lax.axis_index("c")
        s = lax.axis_index("s")
        wid = cc * NS + s

        def idx_start(chunk, bi):
            base = (wid * NCHUNK + chunk) * 3 * C
            pltpu.async_copy(pk_hbm.at[pl.ds(base, 3 * C)], pk_b[bi], si[bi])

        def idx_wait(bi):
            pltpu.make_async_copy(
                pk_hbm.at[pl.ds(0, 3 * C)], pk_b[bi], si[bi]).wait()
            for g in range(0, C, 16):
                row_b[bi][pl.ds(g, 16)] = pk_b[bi][pl.ds(g, 16)]

        def gather_start(bi, br):
            pltpu.async_copy(x_hbm.at[pk_b[bi].at[pl.ds(C, C)]],
                             rows_b[br], sg[br])

        def gather_wait(bi, br):
            pltpu.make_async_copy(x_hbm.at[pk_b[bi].at[pl.ds(C, C)]],
                                  rows_b[br], sg[br]).wait()

        def scatter_start(bi, br):
            pltpu.make_async_copy(
                rows_b[br], acc.at[row_b[bi]], ss[br]).start(add=True)

        def scatter_wait(bi, br):
            pltpu.make_async_copy(
                rows_b[br], acc.at[row_b[bi]], ss[br]).wait()

        def scale(bi, br):
            rv = rows_b[br]
            vv = pk_b[bi]

            @pl.loop(0, C, step=16)
            def _(g):
                val16 = plsc.bitcast(vv[pl.ds(2 * C + g, 16)], jnp.float32)
                for i in range(16):
                    v = val16[i]
                    for j in range(0, D, 16):
                        rv[g + i, pl.ds(j, 16)] = rv[g + i, pl.ds(j, 16)] * v

        @pl.loop(0, C)
        def _(i):
            for j in range(0, D, 16):
                rows_b[0][i, pl.ds(j, 16)] = jnp.zeros((16,), jnp.float32)

        def zero_rows(nrows):
            off = 0
            while off < nrows:
                n = min(C, nrows - off)
                pltpu.sync_copy(rows_b[0].at[pl.ds(0, n)],
                                acc.at[pl.ds(s * RPS + off, n)])
                off += n

        @pl.when(s < NS - 1)
        def _():
            zero_rows(RPS)

        @pl.when(s == NS - 1)
        def _():
            zero_rows(N_NODES - (NS - 1) * RPS)

        for i in range(4):
            idx_start(i, i)
        for i in range(3):
            idx_wait(i)
            gather_start(i, i)
        for i in range(4, 7):
            idx_start(i, i)
        plsc.subcore_barrier()

        def block(c, sw, iw_g, istart):
            bi, br = c % NI, c % NR
            gather_wait(bi, br)
            scale(bi, br)
            scatter_start(bi, br)
            if sw:
                scatter_wait((c - 1) % NI, (c - 1) % NR)
            if iw_g:
                idx_wait((c + 3) % NI)
                gather_start((c + 3) % NI, (c + 3) % NR)
            if istart is not None:
                idx_start(istart, (c + 7) % NI)

        block(0, False, True, 7)
        block(1, True, True, 8)
        block(2, True, True, 9)
        block(3, True, True, 10)
        block(4, True, True, 11)

        @pl.loop(0, (NCHUNK - 5) // 8)
        def _(k):
            c0 = 5 + k * 8
            for j in range(8):
                c = c0 + j
                bi = (5 + j) % NI
                br = (5 + j) % NR
                gather_wait(bi, br)
                scale(bi, br)
                scatter_start(bi, br)
                scatter_wait((4 + j) % NI, (4 + j) % NR)

                @pl.when(c + 3 < NCHUNK)
                def _():
                    idx_wait((j) % NI)
                    gather_start((j) % NI, (j) % NR)

                @pl.when(c + 7 < NCHUNK)
                def _():
                    idx_start(c + 7, (4 + j) % NI)

        scatter_wait((NCHUNK - 1) % NI, (NCHUNK - 1) % NR)
        plsc.subcore_barrier()

        def copy_out(nrows):
            pltpu.sync_copy(acc.at[pl.ds(s * RPS, nrows)],
                            out_hbm.at[cc, pl.ds(s * RPS, nrows)])

        @pl.when(s < NS - 1)
        def _():
            copy_out(RPS)

        @pl.when(s == NS - 1)
        def _():
            copy_out(N_NODES - (NS - 1) * RPS)

    val_i = jax.lax.bitcast_convert_type(val, jnp.int32)
    packed = jnp.stack([row, col, val_i])
    packed = packed.reshape(3, NW, NCHUNK, C)
    packed = packed.transpose(1, 2, 0, 3).reshape(-1)
    return agg(packed, X)


def _tc_linear(partials, weight, bias):
    def body(p_ref, w_ref, b_ref, o_ref):
        h = p_ref[0] + p_ref[1]
        o_ref[...] = (
            jnp.dot(h, w_ref[...], preferred_element_type=jnp.float32)
            + b_ref[...]
        )

    return pl.pallas_call(
        body,
        out_shape=jax.ShapeDtypeStruct((N_NODES, D), jnp.float32),
    )(partials, weight, bias.reshape(1, D))


def kernel(edge_index, edge_values, X, weight, bias):
    partials = _sc_aggregate(edge_index[0], edge_index[1], edge_values, X)
    return _tc_linear(partials, weight, bias)


# speedup vs baseline: 1.0230x; 1.0230x over previous
import functools

import jax
import jax.numpy as jnp
from jax import lax
from jax.experimental import pallas as pl
from jax.experimental.pallas import tpu as pltpu
from jax.experimental.pallas import tpu_sc as plsc

N_NODES = 10000
N_EDGES = 320000
D = 128

NC = 2
NS = 16
NW = NC * NS

C = 80
EPW = N_EDGES // NW
NCHUNK = EPW // C
NI = 8
NR = 4
RPS = 632


def _sc_aggregate(row, col, val, X):
    mesh = plsc.VectorSubcoreMesh(core_axis_name="c", subcore_axis_name="s")

    @functools.partial(
        pl.kernel,
        out_type=jax.ShapeDtypeStruct((NC, N_NODES, D), jnp.float32),
        mesh=mesh,
        scratch_types=(
            [pltpu.VMEM((C,), jnp.int32)] * NI
            + [pltpu.VMEM((C,), jnp.int32)] * NI
            + [pltpu.VMEM((C,), jnp.float32)] * NI
            + [pltpu.VMEM((C, D), jnp.float32)] * NR
            + [pltpu.VMEM_SHARED((N_NODES, D), jnp.float32)]
            + [pltpu.SemaphoreType.DMA] * (NI + 2 * NR)
        ),
    )
    def agg(row_hbm, col_hbm, val_hbm, x_hbm, out_hbm, *refs):
        row_b = refs[0:NI]
        col_b = refs[NI:2 * NI]
        val_b = refs[2 * NI:3 * NI]
        rows_b = refs[3 * NI:3 * NI + NR]
        acc = refs[3 * NI + NR]
        si = refs[3 * NI + NR + 1:3 * NI + NR + 1 + NI]
        sg = refs[3 * NI + NR + 1 + NI:3 * NI + NR + 1 + NI + NR]
        ss = refs[3 * NI + NR + 1 + NI + NR:]

        cc = lax.axis_index("c")
        s = lax.axis_index("s")
        wid = cc * NS + s

        def idx_start(chunk, bi):
            base = wid * EPW + chunk * C
            pltpu.async_copy(row_hbm.at[pl.ds(base, C)], row_b[bi], si[bi])
            pltpu.async_copy(col_hbm.at[pl.ds(base, C)], col_b[bi], si[bi])
            pltpu.async_copy(val_hbm.at[pl.ds(base, C)], val_b[bi], si[bi])

        def idx_wait(bi):
            pltpu.make_async_copy(
                row_hbm.at[pl.ds(0, C)], row_b[bi], si[bi]).wait()
            pltpu.make_async_copy(
                col_hbm.at[pl.ds(0, C)], col_b[bi], si[bi]).wait()
            pltpu.make_async_copy(
                val_hbm.at[pl.ds(0, C)], val_b[bi], si[bi]).wait()

        def gather_start(bi, br):
            pltpu.async_copy(x_hbm.at[col_b[bi]], rows_b[br], sg[br])

        def gather_wait(bi, br):
            pltpu.make_async_copy(
                x_hbm.at[col_b[bi]], rows_b[br], sg[br]).wait()

        def scatter_start(bi, br):
            pltpu.make_async_copy(
                rows_b[br], acc.at[row_b[bi]], ss[br]).start(add=True)

        def scatter_wait(bi, br):
            pltpu.make_async_copy(
                rows_b[br], acc.at[row_b[bi]], ss[br]).wait()

        def scale(bi, br):
            rv = rows_b[br]
            vv = val_b[bi]

            @plsc.parallel_loop(0, C, step=16)
            def _(g):
                val16 = vv[pl.ds(g, 16)]
                for i in range(16):
                    v = val16[i]
                    for j in range(0, D, 16):
                        rv[g + i, pl.ds(j, 16)] = rv[g + i, pl.ds(j, 16)] * v

        @pl.loop(0, C)
        def _(i):
            for j in range(0, D, 16):
                rows_b[0][i, pl.ds(j, 16)] = jnp.zeros((16,), jnp.float32)

        def zero_rows(nrows):
            off = 0
            while off < nrows:
                n = min(C, nrows - off)
                pltpu.sync_copy(rows_b[0].at[pl.ds(0, n)],
                                acc.at[pl.ds(s * RPS + off, n)])
                off += n

        @pl.when(s < NS - 1)
        def _():
            zero_rows(RPS)

        @pl.when(s == NS - 1)
        def _():
            zero_rows(N_NODES - (NS - 1) * RPS)

        for i in range(4):
            idx_start(i, i)
        for i in range(3):
            idx_wait(i)
            gather_start(i, i)
        for i in range(4, 7):
            idx_start(i, i)
        plsc.subcore_barrier()

        def block(c, sw, iw_g, istart):
            bi, br = c % NI, c % NR
            gather_wait(bi, br)
            scale(bi, br)
            scatter_start(bi, br)
            if sw:
                scatter_wait((c - 1) % NI, (c - 1) % NR)
            if iw_g:
                idx_wait((c + 3) % NI)
                gather_start((c + 3) % NI, (c + 3) % NR)
            if istart is not None:
                idx_start(istart, (c + 7) % NI)

        block(0, False, True, 7)
        block(1, True, True, 8)
        block(2, True, True, 9)
        block(3, True, True, 10)
        block(4, True, True, 11)

        @pl.loop(0, (NCHUNK - 5) // 8)
        def _(k):
            c0 = 5 + k * 8
            for j in range(8):
                c = c0 + j
                bi = (5 + j) % NI
                br = (5 + j) % NR
                gather_wait(bi, br)
                scale(bi, br)
                scatter_start(bi, br)
                scatter_wait((4 + j) % NI, (4 + j) % NR)

                @pl.when(c + 3 < NCHUNK)
                def _():
                    idx_wait((j) % NI)
                    gather_start((j) % NI, (j) % NR)

                @pl.when(c + 7 < NCHUNK)
                def _():
                    idx_start(c + 7, (4 + j) % NI)

        scatter_wait((NCHUNK - 1) % NI, (NCHUNK - 1) % NR)
        plsc.subcore_barrier()

        def copy_out(nrows):
            pltpu.sync_copy(acc.at[pl.ds(s * RPS, nrows)],
                            out_hbm.at[cc, pl.ds(s * RPS, nrows)])

        @pl.when(s < NS - 1)
        def _():
            copy_out(RPS)

        @pl.when(s == NS - 1)
        def _():
            copy_out(N_NODES - (NS - 1) * RPS)

    return agg(row, col, val, X)


def _tc_linear(partials, weight, bias):
    def body(p_ref, w_ref, b_ref, o_ref):
        h = p_ref[0] + p_ref[1]
        o_ref[...] = (
            jnp.dot(h, w_ref[...], preferred_element_type=jnp.float32)
            + b_ref[...]
        )

    return pl.pallas_call(
        body,
        out_shape=jax.ShapeDtypeStruct((N_NODES, D), jnp.float32),
    )(partials, weight, bias.reshape(1, D))


def kernel(edge_index, edge_values, X, weight, bias):
    partials = _sc_aggregate(edge_index[0], edge_index[1], edge_values, X)
    return _tc_linear(partials, weight, bias)


# speedup vs baseline: 1.1304x; 1.1050x over previous
import functools

import jax
import jax.numpy as jnp
from jax import lax
from jax.experimental import pallas as pl
from jax.experimental.pallas import tpu as pltpu
from jax.experimental.pallas import tpu_sc as plsc

N_NODES = 10000
N_EDGES = 320000
D = 128

NC = 2
NS = 16
NW = NC * NS

C = 80
EPW = N_EDGES // NW
NCHUNK = EPW // C
NI = 8
NR = 4
RPS = 632


def _sc_aggregate(row, col, val, X):
    mesh = plsc.VectorSubcoreMesh(core_axis_name="c", subcore_axis_name="s")

    @functools.partial(
        pl.kernel,
        out_type=jax.ShapeDtypeStruct((NC, N_NODES, D), jnp.float32),
        mesh=mesh,
        scratch_types=(
            [pltpu.VMEM((C,), jnp.int32)] * NI
            + [pltpu.VMEM((C,), jnp.int32)] * NI
            + [pltpu.VMEM((C,), jnp.float32)] * NI
            + [pltpu.VMEM((C, D), jnp.float32)] * NR
            + [pltpu.VMEM_SHARED((N_NODES, D), jnp.float32)]
            + [pltpu.SemaphoreType.DMA] * (NI + 2 * NR)
        ),
    )
    def agg(row_hbm, col_hbm, val_hbm, x_hbm, out_hbm, *refs):
        row_b = refs[0:NI]
        col_b = refs[NI:2 * NI]
        val_b = refs[2 * NI:3 * NI]
        rows_b = refs[3 * NI:3 * NI + NR]
        acc = refs[3 * NI + NR]
        si = refs[3 * NI + NR + 1:3 * NI + NR + 1 + NI]
        sg = refs[3 * NI + NR + 1 + NI:3 * NI + NR + 1 + NI + NR]
        ss = refs[3 * NI + NR + 1 + NI + NR:]

        cc = lax.axis_index("c")
        s = lax.axis_index("s")
        wid = cc * NS + s

        def idx_start(chunk, bi):
            base = wid * EPW + chunk * C
            pltpu.async_copy(row_hbm.at[pl.ds(base, C)], row_b[bi], si[bi])
            pltpu.async_copy(col_hbm.at[pl.ds(base, C)], col_b[bi], si[bi])
            pltpu.async_copy(val_hbm.at[pl.ds(base, C)], val_b[bi], si[bi])

        def idx_wait(bi):
            pltpu.make_async_copy(
                row_hbm.at[pl.ds(0, C)], row_b[bi], si[bi]).wait()
            pltpu.make_async_copy(
                col_hbm.at[pl.ds(0, C)], col_b[bi], si[bi]).wait()
            pltpu.make_async_copy(
                val_hbm.at[pl.ds(0, C)], val_b[bi], si[bi]).wait()

        def gather_start(bi, br):
            pltpu.async_copy(x_hbm.at[col_b[bi]], rows_b[br], sg[br])

        def gather_wait(bi, br):
            pltpu.make_async_copy(
                x_hbm.at[col_b[bi]], rows_b[br], sg[br]).wait()

        def scatter_start(bi, br):
            pltpu.make_async_copy(
                rows_b[br], acc.at[row_b[bi]], ss[br]).start(add=True)

        def scatter_wait(bi, br):
            pltpu.make_async_copy(
                rows_b[br], acc.at[row_b[bi]], ss[br]).wait()

        def scale(bi, br):
            rv = rows_b[br]
            vv = val_b[bi]

            @pl.loop(0, C, step=16)
            def _(g):
                val16 = vv[pl.ds(g, 16)]
                for i in range(16):
                    v = val16[i]
                    for j in range(0, D, 16):
                        rv[g + i, pl.ds(j, 16)] = rv[g + i, pl.ds(j, 16)] * v

        @pl.loop(0, C)
        def _(i):
            for j in range(0, D, 16):
                rows_b[0][i, pl.ds(j, 16)] = jnp.zeros((16,), jnp.float32)

        def zero_rows(nrows):
            off = 0
            while off < nrows:
                n = min(C, nrows - off)
                pltpu.sync_copy(rows_b[0].at[pl.ds(0, n)],
                                acc.at[pl.ds(s * RPS + off, n)])
                off += n

        @pl.when(s < NS - 1)
        def _():
            zero_rows(RPS)

        @pl.when(s == NS - 1)
        def _():
            zero_rows(N_NODES - (NS - 1) * RPS)

        for i in range(4):
            idx_start(i, i)
        for i in range(3):
            idx_wait(i)
            gather_start(i, i)
        for i in range(4, 7):
            idx_start(i, i)
        plsc.subcore_barrier()

        def block(c, sw, iw_g, istart):
            bi, br = c % NI, c % NR
            gather_wait(bi, br)
            scale(bi, br)
            if sw:
                scatter_wait((c - 1) % NI, (c - 1) % NR)
            scatter_start(bi, br)
            if iw_g:
                idx_wait((c + 3) % NI)
                gather_start((c + 3) % NI, (c + 3) % NR)
            if istart is not None:
                idx_start(istart, (c + 7) % NI)

        block(0, False, True, 7)
        block(1, True, True, 8)
        block(2, True, True, 9)
        block(3, True, True, 10)
        block(4, True, True, 11)

        @pl.loop(0, (NCHUNK - 5) // 8)
        def _(k):
            c0 = 5 + k * 8
            for j in range(8):
                c = c0 + j
                bi = (5 + j) % NI
                br = (5 + j) % NR
                gather_wait(bi, br)
                scale(bi, br)
                scatter_wait((4 + j) % NI, (4 + j) % NR)
                scatter_start(bi, br)

                @pl.when(c + 3 < NCHUNK)
                def _():
                    idx_wait((j) % NI)
                    gather_start((j) % NI, (j) % NR)

                @pl.when(c + 7 < NCHUNK)
                def _():
                    idx_start(c + 7, (4 + j) % NI)

        scatter_wait((NCHUNK - 1) % NI, (NCHUNK - 1) % NR)
        plsc.subcore_barrier()

        def copy_out(nrows):
            pltpu.sync_copy(acc.at[pl.ds(s * RPS, nrows)],
                            out_hbm.at[cc, pl.ds(s * RPS, nrows)])

        @pl.when(s < NS - 1)
        def _():
            copy_out(RPS)

        @pl.when(s == NS - 1)
        def _():
            copy_out(N_NODES - (NS - 1) * RPS)

    return agg(row, col, val, X)


def _tc_linear(partials, weight, bias):
    def body(p_ref, w_ref, b_ref, o_ref):
        h = p_ref[0] + p_ref[1]
        o_ref[...] = (
            jnp.dot(h, w_ref[...], preferred_element_type=jnp.float32)
            + b_ref[...]
        )

    return pl.pallas_call(
        body,
        out_shape=jax.ShapeDtypeStruct((N_NODES, D), jnp.float32),
    )(partials, weight, bias.reshape(1, D))


def kernel(edge_index, edge_values, X, weight, bias):
    partials = _sc_aggregate(edge_index[0], edge_index[1], edge_values, X)
    return _tc_linear(partials, weight, bias)
